# baseline (speedup 1.0000x reference)
import jax
import jax.numpy as jnp
from jax import lax
from jax.experimental import pallas as pl
from jax.experimental.pallas import tpu as pltpu
from jax.experimental.pallas import tpu_sc as plsc

N = 10000
E = 320000
D = 128

NC = 2
NS = 16
N_PAD = 10240
NPT = 640
DH = 64
ECH = 128
SLAB = 8
NCHUNK = 160
NSLAB = NCHUNK // SLAB
EPT = ECH * NCHUNK
E_PAD = EPT * NS
ZROW = N_PAD - 1
NCX = NPT // ECH


def _sc_body(x_hbm, edges_hbm, out_hbm, pp0, pp1, deg_sh,
             eslab0, eslab1, rows0, rows1, gchunk,
             degloc, dinvloc, d2loc, sdloc, val0, val1,
             se0, se1, ss0, ss1, sg0, sg1):
    core = lax.axis_index("c")
    sub = lax.axis_index("s")
    row0 = sub * NPT
    col0 = core * DH

    eslab = (eslab0, eslab1)
    rows = (rows0, rows1)
    val = (val0, val1)
    se = (se0, se1)
    ss = (ss0, ss1)
    sg = (sg0, sg1)

    zero16 = jnp.zeros((16,), jnp.float32)

    def load_slab(slab, sb):
        pltpu.async_copy(
            edges_hbm.at[sub, pl.ds(slab * SLAB, SLAB)], eslab[sb], se[sb])

    def wait_slab(sb):
        pltpu.make_async_copy(
            edges_hbm.at[sub, pl.ds(0, SLAB)], eslab[sb], se[sb]).wait()

    def _zdeg(i, _):
        degloc[pl.ds(i * 16, 16)] = zero16
        return 0
    lax.fori_loop(0, NPT // 16, _zdeg, 0)
    pltpu.sync_copy(degloc, deg_sh.at[pl.ds(row0, NPT)])
    plsc.subcore_barrier()

    load_slab(0, 0)
    load_slab(1, 1)

    def _prep_slab(slab, sb):
        wait_slab(sb)
        pend = [None, None]
        for b8 in range(SLAB):
            bv = b8 % 2
            if pend[bv] is not None:
                pend[bv].wait()
            for k in range(ECH // 16):
                sl = pl.ds(k * 16, 16)
                m = eslab[sb][b8, 0, sl] != eslab[sb][b8, 1, sl]
                val[bv][sl] = jnp.where(m, 1.0, 0.0).astype(jnp.float32)
            pend[bv] = pltpu.async_copy(
                val[bv], deg_sh.at[eslab[sb].at[b8, 1]], ss[bv], add=True)
        pend[0].wait()
        pend[1].wait()
        pl.when(slab + 2 < NSLAB)(lambda: load_slab(slab + 2, sb))

    def _prep(t, _):
        _prep_slab(2 * t, 0)
        _prep_slab(2 * t + 1, 1)
        return 0
    lax.fori_loop(0, NSLAB // 2, _prep, 0)
    plsc.subcore_barrier()

    pltpu.sync_copy(deg_sh.at[pl.ds(row0, NPT)], degloc)
    def _dinv(i, _):
        sl = pl.ds(i * 16, 16)
        d = degloc[sl]
        y = plsc.bitcast(0x5F3759DF - (plsc.bitcast(d, jnp.int32) >> 1),
                         jnp.float32)
        for _ in range(3):
            y = y * (1.5 - 0.5 * d * y * y)
        y = jnp.where(d > 0.0, y, 0.0)
        dinvloc[sl] = y
        d2loc[sl] = y * y
        sdloc[sl] = d * y
        return 0
    lax.fori_loop(0, NPT // 16, _dinv, 0)

    def _zrows1(n, _):
        for q in range(DH // 16):
            rows1[n, pl.ds(q * 16, 16)] = zero16
        return 0
    lax.fori_loop(0, ECH, _zrows1, 0)
    def _init(cix, _):
        nb = row0 + cix * ECH
        pltpu.sync_copy(x_hbm.at[pl.ds(nb, ECH), pl.ds(col0, DH)], gchunk)
        def _gi(i, _):
            dv = dinvloc[pl.ds(cix * ECH + i * 16, 16)]
            for k in range(16):
                wv = jnp.broadcast_to(dv[k], (16,))
                n = i * 16 + k
                for q in range(DH // 16):
                    sl = pl.ds(q * 16, 16)
                    gchunk[n, sl] = wv * gchunk[n, sl]
            return 0
        lax.fori_loop(0, ECH // 16, _gi, 0)
        pltpu.sync_copy(gchunk, pp0.at[pl.ds(nb, ECH), :])
        pltpu.sync_copy(rows1, pp1.at[pl.ds(nb, ECH), :])
        return 0
    lax.fori_loop(0, NCX, _init, 0)
    plsc.subcore_barrier()

    for hop in range(4):
        src_sh = pp0 if hop % 2 == 0 else pp1
        sink_sh = pp1 if hop % 2 == 0 else pp0
        k_out = hop + 1
        oix = {1: 0, 2: 1, 4: 2}.get(k_out)
        pk = 0.5 ** k_out

        load_slab(0, 0)
        load_slab(1, 1)

        def _hop_pair(t, _, src_sh=src_sh, sink_sh=sink_sh):
            wait_slab(0)
            wait_slab(1)
            for sb in range(2):
                for b8 in range(SLAB):
                    for k in range(ECH // 16):
                        sl = pl.ds(k * 16, 16)
                        s16 = eslab[sb][b8, 0, sl]
                        m = s16 != eslab[sb][b8, 1, sl]
                        eslab[sb][b8, 0, sl] = jnp.where(m, s16, ZROW)
            pend_g = [None, None]
            pend_s = [None, None]
            gsrc = [None, None]
            for jj in range(2 * SLAB):
                sb, b8 = jj // SLAB, jj % SLAB
                br = jj % 2
                if pend_s[br] is not None:
                    pend_s[br].wait()
                pend_s[br] = None
                dg = pltpu.async_copy(
                    src_sh.at[eslab[sb].at[b8, 0]], rows[br], sg[br])
                pb = 1 - br
                if pend_g[pb] is not None:
                    pend_g[pb].wait()
                    pend_s[pb] = pltpu.async_copy(
                        rows[pb], sink_sh.at[gsrc[pb]], ss[pb], add=True)
                pend_g[br] = dg
                gsrc[br] = eslab[sb].at[b8, 1]
            lb = (2 * SLAB - 1) % 2
            pend_g[lb].wait()
            pend_s[lb] = pltpu.async_copy(
                rows[lb], sink_sh.at[gsrc[lb]], ss[lb], add=True)
            pend_s[0].wait()
            pend_s[1].wait()
            pl.when(2 * t + 2 < NSLAB)(lambda: load_slab(2 * t + 2, 0))
            pl.when(2 * t + 3 < NSLAB)(lambda: load_slab(2 * t + 3, 1))
            return 0
        lax.fori_loop(0, NSLAB // 2, _hop_pair, 0)
        plsc.subcore_barrier()

        def _xform(cix, _, src_sh=src_sh, sink_sh=sink_sh, oix=oix, pk=pk):
            nb = row0 + cix * ECH
            pltpu.sync_copy(sink_sh.at[pl.ds(nb, ECH), :], rows0)
            pltpu.sync_copy(src_sh.at[pl.ds(nb, ECH), :], gchunk)
            if oix is not None:
                pltpu.sync_copy(
                    x_hbm.at[pl.ds(nb, ECH), pl.ds(col0, DH)], rows1)
            def _tr(i, _):
                base = cix * ECH + i * 16
                d2v = d2loc[pl.ds(base, 16)]
                sdv = sdloc[pl.ds(base, 16)]
                for k in range(16):
                    w2 = jnp.broadcast_to(d2v[k], (16,))
                    n = i * 16 + k
                    for q in range(DH // 16):
                        sl = pl.ds(q * 16, 16)
                        gn = 0.5 * (gchunk[n, sl] + w2 * rows0[n, sl])
                        gchunk[n, sl] = gn
                        if oix is not None:
                            sd = jnp.broadcast_to(sdv[k], (16,))
                            rows0[n, sl] = jnp.where(
                                sd > 0.0, gn * sd, pk * rows1[n, sl])
                return 0
            lax.fori_loop(0, ECH // 16, _tr, 0)
            pltpu.sync_copy(gchunk, sink_sh.at[pl.ds(nb, ECH), :])
            if oix is not None:
                pltpu.sync_copy(
                    rows0, out_hbm.at[oix, pl.ds(nb, ECH), pl.ds(col0, DH)])
            def _zg(n, _):
                for q in range(DH // 16):
                    gchunk[n, pl.ds(q * 16, 16)] = zero16
                return 0
            lax.fori_loop(0, ECH, _zg, 0)
            pltpu.sync_copy(gchunk, src_sh.at[pl.ds(nb, ECH), :])
            return 0
        lax.fori_loop(0, NCX, _xform, 0)
        if hop != 3:
            plsc.subcore_barrier()


def _sc_propagate(x_pad, edges):
    mesh = plsc.VectorSubcoreMesh(core_axis_name="c", subcore_axis_name="s")
    f = pl.kernel(
        _sc_body,
        out_type=jax.ShapeDtypeStruct((3, N_PAD, D), jnp.float32),
        mesh=mesh,
        scratch_types=[
            pltpu.VMEM_SHARED((N_PAD, DH), jnp.float32),
            pltpu.VMEM_SHARED((N_PAD, DH), jnp.float32),
            pltpu.VMEM_SHARED((N_PAD,), jnp.float32),
            pltpu.VMEM((SLAB, 2, ECH), jnp.int32),
            pltpu.VMEM((SLAB, 2, ECH), jnp.int32),
            pltpu.VMEM((ECH, DH), jnp.float32),
            pltpu.VMEM((ECH, DH), jnp.float32),
            pltpu.VMEM((ECH, DH), jnp.float32),
            pltpu.VMEM((NPT,), jnp.float32),
            pltpu.VMEM((NPT,), jnp.float32),
            pltpu.VMEM((NPT,), jnp.float32),
            pltpu.VMEM((NPT,), jnp.float32),
            pltpu.VMEM((ECH,), jnp.float32),
            pltpu.VMEM((ECH,), jnp.float32),
            pltpu.SemaphoreType.DMA,
            pltpu.SemaphoreType.DMA,
            pltpu.SemaphoreType.DMA,
            pltpu.SemaphoreType.DMA,
            pltpu.SemaphoreType.DMA,
            pltpu.SemaphoreType.DMA,
        ],
        compiler_params=pltpu.CompilerParams(use_tc_tiling_on_sc=False,
                                             needs_layout_passes=False),
        name="gcn_sc_propagate",
    )
    return f(x_pad, edges)


def _tc_body(x_ref, h1_ref, h2_ref, h4_ref, w_ref, b_ref, o_ref):
    w = w_ref[...]
    b0 = w[0 * D:1 * D] + w[4 * D:5 * D]
    b1 = w[1 * D:2 * D] - w[4 * D:5 * D] + w[5 * D:6 * D]
    b2 = w[2 * D:3 * D] - w[5 * D:6 * D] + w[6 * D:7 * D]
    b3 = w[3 * D:4 * D] - w[6 * D:7 * D]
    bmat = jnp.concatenate([b0, b1, b2, b3], axis=0)
    xc = jnp.concatenate(
        [x_ref[...], h1_ref[...], h2_ref[...], h4_ref[...]], axis=1)
    o = jnp.dot(xc, bmat, preferred_element_type=jnp.float32) + b_ref[...]
    o_ref[...] = jnp.where(o >= 0.0, o, 0.01 * o)


def _tc_combine(x_pad, h1, h2, h4, W, b):
    blk = 512
    grid = (N_PAD // blk,)
    row_spec = pl.BlockSpec((blk, D), lambda i: (i, 0))
    return pl.pallas_call(
        _tc_body,
        grid=grid,
        in_specs=[
            row_spec, row_spec, row_spec, row_spec,
            pl.BlockSpec((7 * D, D), lambda i: (0, 0)),
            pl.BlockSpec((1, D), lambda i: (0, 0)),
        ],
        out_specs=row_spec,
        out_shape=jax.ShapeDtypeStruct((N_PAD, D), jnp.float32),
    )(x_pad, h1, h2, h4, W, b)


@jax.jit
def kernel(x, edge_index, W, b):
    x_pad = jnp.pad(x, ((0, N_PAD - N), (0, 0)))
    edges = jnp.pad(edge_index, ((0, 0), (0, E_PAD - E)))
    edges = edges.reshape(2, NS, NCHUNK, ECH).transpose(1, 2, 0, 3)
    hks = _sc_propagate(x_pad, edges)
    out = _tc_combine(x_pad, hks[0], hks[1], hks[2], W, b.reshape(1, D))
    return out[:N]


# speedup vs baseline: 1.2599x; 1.2599x over previous
import jax
import jax.numpy as jnp
from jax import lax
from jax.experimental import pallas as pl
from jax.experimental.pallas import tpu as pltpu
from jax.experimental.pallas import tpu_sc as plsc

N = 10000
E = 320000
D = 128

NC = 2
NS = 16
N_PAD = 10240
NPT = 640
DH = 64
ECH = 128
SLAB = 8
NCHUNK = 160
NSLAB = NCHUNK // SLAB
EPT = ECH * NCHUNK
E_PAD = EPT * NS
ZROW = N_PAD - 1
NCX = NPT // ECH


def _sc_body(x_hbm, edges_hbm, out_hbm, pp0, pp1, deg_sh,
             eslab0, eslab1, rows0, rows1, rows2, rows3, rows4,
             degloc, dinvloc, d2loc, sdloc, val0, val1, val2, val3,
             se0, se1, ss0, ss1, ss2, ss3, ss4, sg0, sg1, sg2, sg3, sg4):
    core = lax.axis_index("c")
    sub = lax.axis_index("s")
    row0 = sub * NPT
    col0 = core * DH

    eslab = (eslab0, eslab1)
    rows = (rows0, rows1, rows2, rows3, rows4)
    val = (val0, val1, val2, val3)
    se = (se0, se1)
    ss = (ss0, ss1, ss2, ss3, ss4)
    sg = (sg0, sg1, sg2, sg3, sg4)

    zero16 = jnp.zeros((16,), jnp.float32)

    def load_slab(slab, sb):
        pltpu.async_copy(
            edges_hbm.at[sub, pl.ds(slab * SLAB, SLAB)], eslab[sb], se[sb])

    def wait_slab(sb):
        pltpu.make_async_copy(
            edges_hbm.at[sub, pl.ds(0, SLAB)], eslab[sb], se[sb]).wait()

    def _zdeg(i, _):
        degloc[pl.ds(i * 16, 16)] = zero16
        return 0
    lax.fori_loop(0, NPT // 16, _zdeg, 0)
    pltpu.sync_copy(degloc, deg_sh.at[pl.ds(row0, NPT)])
    plsc.subcore_barrier()

    load_slab(0, 0)
    load_slab(1, 1)

    def _prep_slab(slab, sb):
        wait_slab(sb)
        pend = [None, None, None, None]
        for b8 in range(SLAB):
            bv = b8 % 4
            if pend[bv] is not None:
                pend[bv].wait()
            for k in range(ECH // 16):
                sl = pl.ds(k * 16, 16)
                m = eslab[sb][b8, 0, sl] != eslab[sb][b8, 1, sl]
                val[bv][sl] = jnp.where(m, 1.0, 0.0).astype(jnp.float32)
            pend[bv] = pltpu.async_copy(
                val[bv], deg_sh.at[eslab[sb].at[b8, 1]], ss[bv], add=True)
        for b in range(4):
            if pend[b] is not None:
                pend[b].wait()
        pl.when(slab + 2 < NSLAB)(lambda: load_slab(slab + 2, sb))

    def _prep(t, _):
        _prep_slab(2 * t, 0)
        _prep_slab(2 * t + 1, 1)
        return 0
    lax.fori_loop(0, NSLAB // 2, _prep, 0)
    plsc.subcore_barrier()

    pltpu.sync_copy(deg_sh.at[pl.ds(row0, NPT)], degloc)
    def _dinv(i, _):
        sl = pl.ds(i * 16, 16)
        d = degloc[sl]
        y = plsc.bitcast(0x5F3759DF - (plsc.bitcast(d, jnp.int32) >> 1),
                         jnp.float32)
        for _ in range(3):
            y = y * (1.5 - 0.5 * d * y * y)
        y = jnp.where(d > 0.0, y, 0.0)
        dinvloc[sl] = y
        d2loc[sl] = y * y
        sdloc[sl] = d * y
        return 0
    lax.fori_loop(0, NPT // 16, _dinv, 0)

    def _zrows3(n, _):
        for q in range(DH // 16):
            rows3[n, pl.ds(q * 16, 16)] = zero16
        return 0
    lax.fori_loop(0, ECH, _zrows3, 0)
    NB = (N // ECH) * ECH
    NT = N - NB

    def load_x(nb, dst):
        @pl.when(nb + ECH <= N)
        def _():
            pltpu.sync_copy(x_hbm.at[pl.ds(nb, ECH), pl.ds(col0, DH)],
                            dst.at[pl.ds(0, ECH), :])
        @pl.when(nb == NB)
        def _():
            pltpu.sync_copy(x_hbm.at[pl.ds(nb, NT), pl.ds(col0, DH)],
                            dst.at[pl.ds(0, NT), :])

    def _init(cix, _):
        nb = row0 + cix * ECH
        load_x(nb, rows2)
        def _gi(i, _):
            dv = dinvloc[pl.ds(cix * ECH + i * 16, 16)]
            for k in range(16):
                wv = jnp.broadcast_to(dv[k], (16,))
                n = i * 16 + k
                for q in range(DH // 16):
                    sl = pl.ds(q * 16, 16)
                    rows2[n, sl] = wv * rows2[n, sl]
            return 0
        lax.fori_loop(0, ECH // 16, _gi, 0)
        pltpu.sync_copy(rows2, pp0.at[pl.ds(nb, ECH), :])
        pltpu.sync_copy(rows3, pp1.at[pl.ds(nb, ECH), :])
        return 0
    lax.fori_loop(0, NCX, _init, 0)
    plsc.subcore_barrier()

    for hop in range(4):
        src_sh = pp0 if hop % 2 == 0 else pp1
        sink_sh = pp1 if hop % 2 == 0 else pp0
        k_out = hop + 1
        oix = {1: 0, 2: 1, 4: 2}.get(k_out)
        pk = 0.5 ** k_out

        load_slab(0, 0)
        load_slab(1, 1)

        def _hop_pair(t, _, src_sh=src_sh, sink_sh=sink_sh):
            wait_slab(0)
            wait_slab(1)
            npair = 2 * SLAB
            pend_g = [None] * 5
            pend_s = [None] * 5
            gsrc = [None] * 5
            for jj in range(npair + 1):
                if jj < npair:
                    sb, b8 = jj // SLAB, jj % SLAB
                    br = jj % 5
                    for k in range(ECH // 16):
                        sl = pl.ds(k * 16, 16)
                        s16 = eslab[sb][b8, 0, sl]
                        m = s16 != eslab[sb][b8, 1, sl]
                        eslab[sb][b8, 0, sl] = jnp.where(m, s16, ZROW)
                    if pend_s[br] is not None:
                        pend_s[br].wait()
                        pend_s[br] = None
                    pend_g[br] = pltpu.async_copy(
                        src_sh.at[eslab[sb].at[b8, 0]], rows[br], sg[br])
                    gsrc[br] = eslab[sb].at[b8, 1]
                j2 = jj - 1
                if j2 >= 0:
                    b2 = j2 % 5
                    pend_g[b2].wait()
                    pend_g[b2] = None
                    pend_s[b2] = pltpu.async_copy(
                        rows[b2], sink_sh.at[gsrc[b2]], ss[b2], add=True)
            for b in range(5):
                if pend_s[b] is not None:
                    pend_s[b].wait()
            pl.when(2 * t + 2 < NSLAB)(lambda: load_slab(2 * t + 2, 0))
            pl.when(2 * t + 3 < NSLAB)(lambda: load_slab(2 * t + 3, 1))
            return 0
        lax.fori_loop(0, NSLAB // 2, _hop_pair, 0)
        plsc.subcore_barrier()

        def _zr3(n, _):
            for q in range(DH // 16):
                rows3[n, pl.ds(q * 16, 16)] = zero16
            return 0
        lax.fori_loop(0, ECH, _zr3, 0)

        def _xform(cix, _, src_sh=src_sh, sink_sh=sink_sh, oix=oix, pk=pk):
            nb = row0 + cix * ECH
            dl = [pltpu.async_copy(sink_sh.at[pl.ds(nb, ECH), :], rows0, sg0),
                  pltpu.async_copy(src_sh.at[pl.ds(nb, ECH), :], rows2, sg1)]
            for d in dl:
                d.wait()
            if oix is not None:
                load_x(nb, rows1)
            def _tr(i, _):
                base = cix * ECH + i * 16
                d2v = d2loc[pl.ds(base, 16)]
                sdv = sdloc[pl.ds(base, 16)]
                for k in range(16):
                    w2 = jnp.broadcast_to(d2v[k], (16,))
                    n = i * 16 + k
                    for q in range(DH // 16):
                        sl = pl.ds(q * 16, 16)
                        gn = 0.5 * (rows2[n, sl] + w2 * rows0[n, sl])
                        rows2[n, sl] = gn
                        if oix is not None:
                            sd = jnp.broadcast_to(sdv[k], (16,))
                            rows0[n, sl] = jnp.where(
                                sd > 0.0, gn * sd, pk * rows1[n, sl])
                return 0
            lax.fori_loop(0, ECH // 16, _tr, 0)
            ds_ = [pltpu.async_copy(rows2, sink_sh.at[pl.ds(nb, ECH), :], ss0),
                   pltpu.async_copy(rows3, src_sh.at[pl.ds(nb, ECH), :], ss1)]
            if oix is not None:
                ds_.append(pltpu.async_copy(
                    rows0, out_hbm.at[oix, pl.ds(nb, ECH), pl.ds(col0, DH)],
                    ss2))
            for d in ds_:
                d.wait()
            return 0
        lax.fori_loop(0, NCX, _xform, 0)
        if hop != 3:
            plsc.subcore_barrier()


def _sc_propagate(x_pad, edges):
    mesh = plsc.VectorSubcoreMesh(core_axis_name="c", subcore_axis_name="s")
    f = pl.kernel(
        _sc_body,
        out_type=jax.ShapeDtypeStruct((3, N_PAD, D), jnp.float32),
        mesh=mesh,
        scratch_types=[
            pltpu.VMEM_SHARED((N_PAD, DH), jnp.float32),
            pltpu.VMEM_SHARED((N_PAD, DH), jnp.float32),
            pltpu.VMEM_SHARED((N_PAD,), jnp.float32),
            pltpu.VMEM((SLAB, 2, ECH), jnp.int32),
            pltpu.VMEM((SLAB, 2, ECH), jnp.int32),
            pltpu.VMEM((ECH, DH), jnp.float32),
            pltpu.VMEM((ECH, DH), jnp.float32),
            pltpu.VMEM((ECH, DH), jnp.float32),
            pltpu.VMEM((ECH, DH), jnp.float32),
            pltpu.VMEM((ECH, DH), jnp.float32),
            pltpu.VMEM((NPT,), jnp.float32),
            pltpu.VMEM((NPT,), jnp.float32),
            pltpu.VMEM((NPT,), jnp.float32),
            pltpu.VMEM((NPT,), jnp.float32),
            pltpu.VMEM((ECH,), jnp.float32),
            pltpu.VMEM((ECH,), jnp.float32),
            pltpu.VMEM((ECH,), jnp.float32),
            pltpu.VMEM((ECH,), jnp.float32),
            pltpu.SemaphoreType.DMA,
            pltpu.SemaphoreType.DMA,
            pltpu.SemaphoreType.DMA,
            pltpu.SemaphoreType.DMA,
            pltpu.SemaphoreType.DMA,
            pltpu.SemaphoreType.DMA,
            pltpu.SemaphoreType.DMA,
            pltpu.SemaphoreType.DMA,
            pltpu.SemaphoreType.DMA,
            pltpu.SemaphoreType.DMA,
            pltpu.SemaphoreType.DMA,
            pltpu.SemaphoreType.DMA,
        ],
        compiler_params=pltpu.CompilerParams(use_tc_tiling_on_sc=False,
                                             needs_layout_passes=False),
        name="gcn_sc_propagate",
    )
    return f(x_pad, edges)


def _tc_body(x_ref, h1_ref, h2_ref, h4_ref, w_ref, b_ref, o_ref):
    w = w_ref[...]
    b0 = w[0 * D:1 * D] + w[4 * D:5 * D]
    b1 = w[1 * D:2 * D] - w[4 * D:5 * D] + w[5 * D:6 * D]
    b2 = w[2 * D:3 * D] - w[5 * D:6 * D] + w[6 * D:7 * D]
    b3 = w[3 * D:4 * D] - w[6 * D:7 * D]
    bmat = jnp.concatenate([b0, b1, b2, b3], axis=0)
    xc = jnp.concatenate(
        [x_ref[...], h1_ref[0], h2_ref[0], h4_ref[0]], axis=1)
    o = jnp.dot(xc, bmat, precision=jax.lax.Precision.HIGHEST,
                preferred_element_type=jnp.float32) + b_ref[...]
    o_ref[...] = jnp.where(o >= 0.0, o, 0.01 * o)


def _tc_combine(x, hks, W, b):
    blk = 400
    grid = (N // blk,)
    row_spec = pl.BlockSpec((blk, D), lambda i: (i, 0))
    def hspec(k):
        return pl.BlockSpec((1, blk, D), lambda i, k=k: (k, i, 0))
    return pl.pallas_call(
        _tc_body,
        grid=grid,
        in_specs=[
            row_spec, hspec(0), hspec(1), hspec(2),
            pl.BlockSpec((7 * D, D), lambda i: (0, 0)),
            pl.BlockSpec((1, D), lambda i: (0, 0)),
        ],
        out_specs=row_spec,
        out_shape=jax.ShapeDtypeStruct((N, D), jnp.float32),
    )(x, hks, hks, hks, W, b)


@jax.jit
def kernel(x, edge_index, W, b):
    edges = jnp.pad(edge_index, ((0, 0), (0, E_PAD - E)))
    edges = edges.reshape(2, NS, NCHUNK, ECH).transpose(1, 2, 0, 3)
    hks = _sc_propagate(x, edges)
    return _tc_combine(x, hks, W, b.reshape(1, D))
